# baseline (speedup 1.0000x reference)
import functools

import jax
import jax.numpy as jnp
from jax import lax
from jax.experimental import pallas as pl
from jax.experimental.pallas import tpu as pltpu
from jax.experimental.pallas import tpu_sc as plsc

N = 10000
NP = 10240
D = 128
H = 64
G = 64
E = 320000
EP = 327680
DUMMY = 10100

NSUB = 16
ROWS_PER_TILE = NP // NSUB
AGG_CHUNK = 128
AGG_CHUNKS_PER_TILE = EP // (2 * NSUB) // AGG_CHUNK
DEG_CHUNKS_PER_TILE = EP // (2 * NSUB) // AGG_CHUNK

BLK = 512
NBLK = NP // BLK

@functools.cache
def _sc_mesh():
    return plsc.VectorSubcoreMesh(core_axis_name="c", subcore_axis_name="s",
                                  num_cores=2, num_subcores=NSUB)


INIT_CHUNK = 128
INIT_STEPS = ROWS_PER_TILE // INIT_CHUNK


def _deg_body(dst_ref, zeros_ref, ones_ref, out_ref, shared, didx, onesv, stage):
    c = lax.axis_index("c")
    s = lax.axis_index("s")
    base = s * ROWS_PER_TILE
    tile = c * NSUB + s
    pltpu.sync_copy(ones_ref, onesv)
    pltpu.sync_copy(dst_ref.at[pl.ds(tile * DEG_CHUNKS_PER_TILE,
                                     DEG_CHUNKS_PER_TILE)], didx)
    pltpu.sync_copy(zeros_ref, stage)

    @pl.loop(0, INIT_STEPS)
    def _(j):
        pltpu.sync_copy(stage, shared.at[pl.ds(base + j * INIT_CHUNK, INIT_CHUNK)])

    plsc.subcore_barrier()

    @pl.loop(0, DEG_CHUNKS_PER_TILE, unroll=1)
    def _(i):
        pltpu.sync_copy(onesv, shared.at[didx.at[i]], add=True)

    plsc.subcore_barrier()

    obase = c * NP + base

    @pl.loop(0, INIT_STEPS)
    def _(j):
        pltpu.sync_copy(shared.at[pl.ds(base + j * INIT_CHUNK, INIT_CHUNK)], stage)
        pltpu.sync_copy(stage, out_ref.at[pl.ds(obase + j * INIT_CHUNK, INIT_CHUNK)])


@functools.cache
def _deg_kernel():
    return pl.kernel(
        _deg_body,
        out_type=jax.ShapeDtypeStruct((2 * NP, D), jnp.float32),
        mesh=_sc_mesh(),
        scratch_types=[pltpu.VMEM_SHARED((NP, D), jnp.float32),
                       pltpu.VMEM((DEG_CHUNKS_PER_TILE, AGG_CHUNK), jnp.int32),
                       pltpu.VMEM((AGG_CHUNK, D), jnp.float32),
                       pltpu.VMEM((INIT_CHUNK, D), jnp.float32)])


PHASE = 40
SPLIT0 = 160
SPLIT1 = 2 * AGG_CHUNKS_PER_TILE - SPLIT0
TOTAL0 = SPLIT0 * NSUB


def _agg_pipeline(g_ref, src_ref, dst_ref, shared, sidx, didx,
                  rows0, rows1, sem0, sem1, s, cstart, nchunks):
    def gather(chunk, buf, sem):
        pltpu.async_copy(g_ref.at[sidx.at[chunk]], buf, sem)

    def gwait(chunk, buf, sem):
        pltpu.make_async_copy(g_ref.at[sidx.at[chunk]], buf, sem).wait()

    def scat(chunk, buf):
        pltpu.sync_copy(buf, shared.at[didx.at[chunk]], add=True)

    for p in range(nchunks // PHASE):
        cbase = cstart + s * nchunks + p * PHASE
        pltpu.sync_copy(src_ref.at[pl.ds(cbase, PHASE)], sidx)
        pltpu.sync_copy(dst_ref.at[pl.ds(cbase, PHASE)], didx)

        gather(0, rows0, sem0)

        @pl.loop(0, PHASE // 2 - 1, unroll=1)
        def _(j):
            a = 2 * j
            gather(a + 1, rows1, sem1)
            gwait(a, rows0, sem0)
            scat(a, rows0)
            gather(a + 2, rows0, sem0)
            gwait(a + 1, rows1, sem1)
            scat(a + 1, rows1)

        gather(PHASE - 1, rows1, sem1)
        gwait(PHASE - 2, rows0, sem0)
        scat(PHASE - 2, rows0)
        gwait(PHASE - 1, rows1, sem1)
        scat(PHASE - 1, rows1)


def _agg_body(g_ref, src_ref, dst_ref, out_ref, shared,
              sidx, didx, rows0, rows1, sem0, sem1):
    c = lax.axis_index("c")
    s = lax.axis_index("s")
    base = s * ROWS_PER_TILE

    @pl.loop(0, INIT_STEPS)
    def _(j):
        row = base + j * INIT_CHUNK
        pltpu.sync_copy(g_ref.at[pl.ds(row, INIT_CHUNK)], rows0)
        pltpu.sync_copy(rows0, shared.at[pl.ds(row, INIT_CHUNK)])

    plsc.subcore_barrier()

    @pl.when(c == 0)
    def _():
        _agg_pipeline(g_ref, src_ref, dst_ref, shared, sidx, didx,
                      rows0, rows1, sem0, sem1, s, 0, SPLIT0)

    @pl.when(c == 1)
    def _():
        _agg_pipeline(g_ref, src_ref, dst_ref, shared, sidx, didx,
                      rows0, rows1, sem0, sem1, s, TOTAL0, SPLIT1)

    plsc.subcore_barrier()

    obase = c * NP + base

    @pl.loop(0, INIT_STEPS)
    def _(j):
        pltpu.sync_copy(shared.at[pl.ds(base + j * INIT_CHUNK, INIT_CHUNK)], rows0)
        pltpu.sync_copy(rows0, out_ref.at[pl.ds(obase + j * INIT_CHUNK, INIT_CHUNK)])


@functools.cache
def _agg_kernel():
    return pl.kernel(
        _agg_body,
        out_type=jax.ShapeDtypeStruct((2 * NP, D), jnp.float32),
        mesh=_sc_mesh(),
        scratch_types=[pltpu.VMEM_SHARED((NP, D), jnp.float32),
                       pltpu.VMEM((PHASE, AGG_CHUNK), jnp.int32),
                       pltpu.VMEM((PHASE, AGG_CHUNK), jnp.int32),
                       pltpu.VMEM((AGG_CHUNK, D), jnp.float32),
                       pltpu.VMEM((AGG_CHUNK, D), jnp.float32),
                       pltpu.SemaphoreType.DMA,
                       pltpu.SemaphoreType.DMA])


def _mm1_body(x_ref, w_ref, d0_ref, d1_ref, g_ref, dv_ref):
    deg = 1.0 + d0_ref[:, 0:1] + d1_ref[:, 0:1]
    dinv = lax.rsqrt(deg)
    h = jnp.dot(x_ref[...], w_ref[...], preferred_element_type=jnp.float32)
    g_ref[...] = h * dinv
    dv_ref[...] = jnp.broadcast_to(dinv, (BLK, 16))


def _mm1(x, W1, deg):
    return pl.pallas_call(
        _mm1_body,
        grid=(NBLK,),
        in_specs=[pl.BlockSpec((BLK, D), lambda i: (i, 0)),
                  pl.BlockSpec((D, D), lambda i: (0, 0)),
                  pl.BlockSpec((BLK, D), lambda i: (i, 0)),
                  pl.BlockSpec((BLK, D), lambda i: (i + NBLK, 0))],
        out_specs=[pl.BlockSpec((BLK, D), lambda i: (i, 0)),
                   pl.BlockSpec((BLK, 16), lambda i: (i, 0))],
        out_shape=[jax.ShapeDtypeStruct((NP, D), jnp.float32),
                   jax.ShapeDtypeStruct((NP, 16), jnp.float32)],
    )(x, W1, deg, deg)


def _mm2_body(p0_ref, p1_ref, g1_ref, dv_ref, w_ref, b_ref, g_ref):
    dinv = dv_ref[:, 0:1]
    agg = p0_ref[...] + p1_ref[...] - g1_ref[...]
    f = jnp.maximum(dinv * agg + b_ref[...], 0.0)
    h = jnp.dot(f, w_ref[...], preferred_element_type=jnp.float32)
    g_ref[...] = h * dinv


def _mm2(p, g1, dinv16, W2, b1):
    return pl.pallas_call(
        _mm2_body,
        grid=(NBLK,),
        in_specs=[pl.BlockSpec((BLK, D), lambda i: (i, 0)),
                  pl.BlockSpec((BLK, D), lambda i: (i + NBLK, 0)),
                  pl.BlockSpec((BLK, D), lambda i: (i, 0)),
                  pl.BlockSpec((BLK, 16), lambda i: (i, 0)),
                  pl.BlockSpec((D, D), lambda i: (0, 0)),
                  pl.BlockSpec((1, D), lambda i: (0, 0))],
        out_specs=pl.BlockSpec((BLK, D), lambda i: (i, 0)),
        out_shape=jax.ShapeDtypeStruct((NP, D), jnp.float32),
    )(p, p, g1, dinv16, W2, b1)


def _pool_body(p0_ref, p1_ref, g2_ref, dv_ref, b_ref, batch_ref, out_ref,
               acc_ref, cnt_ref):
    i = pl.program_id(0)

    @pl.when(i == 0)
    def _():
        acc_ref[...] = jnp.zeros((G, D), jnp.float32)
        cnt_ref[...] = jnp.zeros((G, D), jnp.float32)

    dinv = dv_ref[:, 0:1]
    f = dinv * (p0_ref[...] + p1_ref[...] - g2_ref[...]) + b_ref[...]
    b = batch_ref[0]
    gids = lax.broadcasted_iota(jnp.int32, (G, BLK), 0)
    oh = (gids == jnp.broadcast_to(b, (G, BLK))).astype(jnp.float32)
    acc_ref[...] += jnp.dot(oh, f, preferred_element_type=jnp.float32)
    cnt_ref[...] += jnp.broadcast_to(
        jnp.sum(oh, axis=1, keepdims=True), (G, D))

    @pl.when(i == NBLK - 1)
    def _():
        pooled = acc_ref[...] / jnp.maximum(cnt_ref[...], 1.0)
        m = jnp.max(pooled, axis=0, keepdims=True)
        z = pooled - m
        lse = jnp.log(jnp.sum(jnp.exp(z), axis=0, keepdims=True))
        out_ref[...] = z - lse


def _pool(q, g2, dinv16, b2, batch3):
    return pl.pallas_call(
        _pool_body,
        grid=(NBLK,),
        in_specs=[pl.BlockSpec((BLK, D), lambda i: (i, 0)),
                  pl.BlockSpec((BLK, D), lambda i: (i + NBLK, 0)),
                  pl.BlockSpec((BLK, D), lambda i: (i, 0)),
                  pl.BlockSpec((BLK, 16), lambda i: (i, 0)),
                  pl.BlockSpec((1, D), lambda i: (0, 0)),
                  pl.BlockSpec((1, 1, BLK), lambda i: (i, 0, 0))],
        out_specs=pl.BlockSpec((G, D), lambda i: (0, 0)),
        out_shape=jax.ShapeDtypeStruct((G, D), jnp.float32),
        scratch_shapes=[pltpu.VMEM((G, D), jnp.float32),
                        pltpu.VMEM((G, D), jnp.float32)],
    )(q, q, g2, dinv16, b2, batch3)


def kernel(x, edge_index, batch, W1, b1, W2, b2):
    src = edge_index[0].astype(jnp.int32)
    dst = edge_index[1].astype(jnp.int32)
    srcp = jnp.full((EP,), DUMMY, jnp.int32).at[:E].set(src).reshape(EP // AGG_CHUNK, AGG_CHUNK)
    dstp = jnp.full((EP,), DUMMY, jnp.int32).at[:E].set(dst).reshape(EP // AGG_CHUNK, AGG_CHUNK)
    xp = jnp.zeros((NP, D), jnp.float32).at[:N].set(x)
    batchp = jnp.full((NP,), G + 1, jnp.int32).at[:N].set(batch.astype(jnp.int32))
    batch3 = batchp.reshape(NBLK, 1, BLK)
    zerosC = jnp.zeros((INIT_CHUNK, D), jnp.float32)
    onesC = jnp.ones((AGG_CHUNK, D), jnp.float32)
    b1r = b1.reshape(1, D)
    b2r = b2.reshape(1, D)

    deg = _deg_kernel()(dstp, zerosC, onesC)
    g1, dinv16 = _mm1(xp, W1, deg)
    p = _agg_kernel()(g1, srcp, dstp)
    g2 = _mm2(p, g1, dinv16, W2, b1r)
    q = _agg_kernel()(g2, srcp, dstp)
    return _pool(q, g2, dinv16, b2r, batch3)


# speedup vs baseline: 1.1300x; 1.1300x over previous
import functools

import jax
import jax.numpy as jnp
from jax import lax
from jax.experimental import pallas as pl
from jax.experimental.pallas import tpu as pltpu
from jax.experimental.pallas import tpu_sc as plsc

N = 10000
NP = 10240
D = 128
H = 64
G = 64
E = 320000
EP = 327680
DUMMY = 10100

NSUB = 16
ROWS_PER_TILE = NP // NSUB
AGG_CHUNK = 128
AGG_CHUNKS_PER_TILE = EP // (2 * NSUB) // AGG_CHUNK
DEG_CHUNKS_PER_TILE = EP // (2 * NSUB) // AGG_CHUNK

BLK = 512
NBLK = NP // BLK

@functools.cache
def _sc_mesh():
    return plsc.VectorSubcoreMesh(core_axis_name="c", subcore_axis_name="s",
                                  num_cores=2, num_subcores=NSUB)


INIT_CHUNK = 128
INIT_STEPS = ROWS_PER_TILE // INIT_CHUNK


def _deg_body(dst_ref, zeros_ref, ones_ref, out_ref, shared, didx, onesv, stage):
    c = lax.axis_index("c")
    s = lax.axis_index("s")
    base = s * ROWS_PER_TILE
    tile = c * NSUB + s
    pltpu.sync_copy(ones_ref, onesv)
    pltpu.sync_copy(dst_ref.at[pl.ds(tile * DEG_CHUNKS_PER_TILE,
                                     DEG_CHUNKS_PER_TILE)], didx)
    pltpu.sync_copy(zeros_ref, stage)

    @pl.loop(0, INIT_STEPS)
    def _(j):
        pltpu.sync_copy(stage, shared.at[pl.ds(base + j * INIT_CHUNK, INIT_CHUNK)])

    plsc.subcore_barrier()

    @pl.loop(0, DEG_CHUNKS_PER_TILE, unroll=1)
    def _(i):
        pltpu.sync_copy(onesv, shared.at[didx.at[i]], add=True)

    plsc.subcore_barrier()

    obase = c * NP + base

    @pl.loop(0, INIT_STEPS)
    def _(j):
        pltpu.sync_copy(shared.at[pl.ds(base + j * INIT_CHUNK, INIT_CHUNK)], stage)
        pltpu.sync_copy(stage, out_ref.at[pl.ds(obase + j * INIT_CHUNK, INIT_CHUNK)])


@functools.cache
def _deg_kernel():
    return pl.kernel(
        _deg_body,
        out_type=jax.ShapeDtypeStruct((2 * NP, D), jnp.float32),
        mesh=_sc_mesh(),
        scratch_types=[pltpu.VMEM_SHARED((NP, D), jnp.float32),
                       pltpu.VMEM((DEG_CHUNKS_PER_TILE, AGG_CHUNK), jnp.int32),
                       pltpu.VMEM((AGG_CHUNK, D), jnp.float32),
                       pltpu.VMEM((INIT_CHUNK, D), jnp.float32)])


PHASE = 40
SPLIT0 = 120
SPLIT1 = 2 * AGG_CHUNKS_PER_TILE - SPLIT0
TOTAL0 = SPLIT0 * NSUB


def _agg_pipeline(g_ref, src_ref, dst_ref, shared, sidx, didx,
                  rows0, rows1, sem0, sem1, sem2, sem3, s, cstart, nchunks):
    def gather(chunk, buf, sem):
        pltpu.async_copy(g_ref.at[sidx.at[chunk]], buf, sem)

    def gwait(chunk, buf, sem):
        pltpu.make_async_copy(g_ref.at[sidx.at[chunk]], buf, sem).wait()

    def scat(chunk, buf, sem):
        pltpu.async_copy(buf, shared.at[didx.at[chunk]], sem, add=True)

    def swait(chunk, buf, sem):
        pltpu.make_async_copy(buf, shared.at[didx.at[chunk]], sem).wait()

    for p in range(nchunks // PHASE):
        cbase = cstart + s * nchunks + p * PHASE
        pltpu.sync_copy(src_ref.at[pl.ds(cbase, PHASE)], sidx)
        pltpu.sync_copy(dst_ref.at[pl.ds(cbase, PHASE)], didx)

        gather(0, rows0, sem0)
        gather(1, rows1, sem1)

        @pl.loop(0, PHASE // 2 - 1, unroll=1)
        def _(j):
            a = 2 * j
            gwait(a, rows0, sem0)
            scat(a, rows0, sem2)
            gwait(a + 1, rows1, sem1)
            scat(a + 1, rows1, sem3)
            swait(a, rows0, sem2)
            gather(a + 2, rows0, sem0)
            swait(a + 1, rows1, sem3)
            gather(a + 3, rows1, sem1)

        a = PHASE - 2
        gwait(a, rows0, sem0)
        scat(a, rows0, sem2)
        gwait(a + 1, rows1, sem1)
        scat(a + 1, rows1, sem3)
        swait(a, rows0, sem2)
        swait(a + 1, rows1, sem3)


def _agg_body(g_ref, src_ref, dst_ref, out_ref, shared,
              sidx, didx, rows0, rows1, sem0, sem1, sem2, sem3):
    c = lax.axis_index("c")
    s = lax.axis_index("s")
    base = s * ROWS_PER_TILE

    @pl.loop(0, INIT_STEPS)
    def _(j):
        row = base + j * INIT_CHUNK
        pltpu.sync_copy(g_ref.at[pl.ds(row, INIT_CHUNK)], rows0)
        pltpu.sync_copy(rows0, shared.at[pl.ds(row, INIT_CHUNK)])

    plsc.subcore_barrier()

    @pl.when(c == 0)
    def _():
        _agg_pipeline(g_ref, src_ref, dst_ref, shared, sidx, didx,
                      rows0, rows1, sem0, sem1, sem2, sem3, s, 0, SPLIT0)

    @pl.when(c == 1)
    def _():
        _agg_pipeline(g_ref, src_ref, dst_ref, shared, sidx, didx,
                      rows0, rows1, sem0, sem1, sem2, sem3, s, TOTAL0, SPLIT1)

    plsc.subcore_barrier()

    obase = c * NP + base

    @pl.loop(0, INIT_STEPS)
    def _(j):
        pltpu.sync_copy(shared.at[pl.ds(base + j * INIT_CHUNK, INIT_CHUNK)], rows0)
        pltpu.sync_copy(rows0, out_ref.at[pl.ds(obase + j * INIT_CHUNK, INIT_CHUNK)])


@functools.cache
def _agg_kernel():
    return pl.kernel(
        _agg_body,
        out_type=jax.ShapeDtypeStruct((2 * NP, D), jnp.float32),
        mesh=_sc_mesh(),
        scratch_types=[pltpu.VMEM_SHARED((NP, D), jnp.float32),
                       pltpu.VMEM((PHASE, AGG_CHUNK), jnp.int32),
                       pltpu.VMEM((PHASE, AGG_CHUNK), jnp.int32),
                       pltpu.VMEM((AGG_CHUNK, D), jnp.float32),
                       pltpu.VMEM((AGG_CHUNK, D), jnp.float32),
                       pltpu.SemaphoreType.DMA,
                       pltpu.SemaphoreType.DMA,
                       pltpu.SemaphoreType.DMA,
                       pltpu.SemaphoreType.DMA])


def _mm1_body(x_ref, w_ref, d0_ref, d1_ref, g_ref, dv_ref):
    deg = 1.0 + d0_ref[:, 0:1] + d1_ref[:, 0:1]
    dinv = lax.rsqrt(deg)
    h = jnp.dot(x_ref[...], w_ref[...], preferred_element_type=jnp.float32)
    g_ref[...] = h * dinv
    dv_ref[...] = jnp.broadcast_to(dinv, (BLK, 16))


def _mm1(x, W1, deg):
    return pl.pallas_call(
        _mm1_body,
        grid=(NBLK,),
        in_specs=[pl.BlockSpec((BLK, D), lambda i: (i, 0)),
                  pl.BlockSpec((D, D), lambda i: (0, 0)),
                  pl.BlockSpec((BLK, D), lambda i: (i, 0)),
                  pl.BlockSpec((BLK, D), lambda i: (i + NBLK, 0))],
        out_specs=[pl.BlockSpec((BLK, D), lambda i: (i, 0)),
                   pl.BlockSpec((BLK, 16), lambda i: (i, 0))],
        out_shape=[jax.ShapeDtypeStruct((NP, D), jnp.float32),
                   jax.ShapeDtypeStruct((NP, 16), jnp.float32)],
    )(x, W1, deg, deg)


def _mm2_body(p0_ref, p1_ref, g1_ref, dv_ref, w_ref, b_ref, g_ref):
    dinv = dv_ref[:, 0:1]
    agg = p0_ref[...] + p1_ref[...] - g1_ref[...]
    f = jnp.maximum(dinv * agg + b_ref[...], 0.0)
    h = jnp.dot(f, w_ref[...], preferred_element_type=jnp.float32)
    g_ref[...] = h * dinv


def _mm2(p, g1, dinv16, W2, b1):
    return pl.pallas_call(
        _mm2_body,
        grid=(NBLK,),
        in_specs=[pl.BlockSpec((BLK, D), lambda i: (i, 0)),
                  pl.BlockSpec((BLK, D), lambda i: (i + NBLK, 0)),
                  pl.BlockSpec((BLK, D), lambda i: (i, 0)),
                  pl.BlockSpec((BLK, 16), lambda i: (i, 0)),
                  pl.BlockSpec((D, D), lambda i: (0, 0)),
                  pl.BlockSpec((1, D), lambda i: (0, 0))],
        out_specs=pl.BlockSpec((BLK, D), lambda i: (i, 0)),
        out_shape=jax.ShapeDtypeStruct((NP, D), jnp.float32),
    )(p, p, g1, dinv16, W2, b1)


def _pool_body(p0_ref, p1_ref, g2_ref, dv_ref, b_ref, batch_ref, out_ref,
               acc_ref, cnt_ref):
    i = pl.program_id(0)

    @pl.when(i == 0)
    def _():
        acc_ref[...] = jnp.zeros((G, D), jnp.float32)
        cnt_ref[...] = jnp.zeros((G, D), jnp.float32)

    dinv = dv_ref[:, 0:1]
    f = dinv * (p0_ref[...] + p1_ref[...] - g2_ref[...]) + b_ref[...]
    b = batch_ref[0]
    gids = lax.broadcasted_iota(jnp.int32, (G, BLK), 0)
    oh = (gids == jnp.broadcast_to(b, (G, BLK))).astype(jnp.float32)
    acc_ref[...] += jnp.dot(oh, f, preferred_element_type=jnp.float32)
    cnt_ref[...] += jnp.broadcast_to(
        jnp.sum(oh, axis=1, keepdims=True), (G, D))

    @pl.when(i == NBLK - 1)
    def _():
        pooled = acc_ref[...] / jnp.maximum(cnt_ref[...], 1.0)
        m = jnp.max(pooled, axis=0, keepdims=True)
        z = pooled - m
        lse = jnp.log(jnp.sum(jnp.exp(z), axis=0, keepdims=True))
        out_ref[...] = z - lse


def _pool(q, g2, dinv16, b2, batch3):
    return pl.pallas_call(
        _pool_body,
        grid=(NBLK,),
        in_specs=[pl.BlockSpec((BLK, D), lambda i: (i, 0)),
                  pl.BlockSpec((BLK, D), lambda i: (i + NBLK, 0)),
                  pl.BlockSpec((BLK, D), lambda i: (i, 0)),
                  pl.BlockSpec((BLK, 16), lambda i: (i, 0)),
                  pl.BlockSpec((1, D), lambda i: (0, 0)),
                  pl.BlockSpec((1, 1, BLK), lambda i: (i, 0, 0))],
        out_specs=pl.BlockSpec((G, D), lambda i: (0, 0)),
        out_shape=jax.ShapeDtypeStruct((G, D), jnp.float32),
        scratch_shapes=[pltpu.VMEM((G, D), jnp.float32),
                        pltpu.VMEM((G, D), jnp.float32)],
    )(q, q, g2, dinv16, b2, batch3)


def kernel(x, edge_index, batch, W1, b1, W2, b2):
    src = edge_index[0].astype(jnp.int32)
    dst = edge_index[1].astype(jnp.int32)
    srcp = jnp.full((EP,), DUMMY, jnp.int32).at[:E].set(src).reshape(EP // AGG_CHUNK, AGG_CHUNK)
    dstp = jnp.full((EP,), DUMMY, jnp.int32).at[:E].set(dst).reshape(EP // AGG_CHUNK, AGG_CHUNK)
    xp = jnp.zeros((NP, D), jnp.float32).at[:N].set(x)
    batchp = jnp.full((NP,), G + 1, jnp.int32).at[:N].set(batch.astype(jnp.int32))
    batch3 = batchp.reshape(NBLK, 1, BLK)
    zerosC = jnp.zeros((INIT_CHUNK, D), jnp.float32)
    onesC = jnp.ones((AGG_CHUNK, D), jnp.float32)
    b1r = b1.reshape(1, D)
    b2r = b2.reshape(1, D)

    deg = _deg_kernel()(dstp, zerosC, onesC)
    g1, dinv16 = _mm1(xp, W1, deg)
    p = _agg_kernel()(g1, srcp, dstp)
    g2 = _mm2(p, g1, dinv16, W2, b1r)
    q = _agg_kernel()(g2, srcp, dstp)
    return _pool(q, g2, dinv16, b2r, batch3)
